# Initial kernel scaffold; baseline (speedup 1.0000x reference)
import jax
import jax.numpy as jnp
from jax.experimental import pallas as pl


def kernel(emissions, tags, token_sizes, transitions, start_transitions, end_transitions):
    raise NotImplementedError("write your pallas kernel here")


# speedup vs baseline: 44.6459x; 44.6459x over previous
import functools

import jax
import jax.numpy as jnp
from jax.experimental import pallas as pl

T, B, C, K = 512, 16, 4, 64
CK = C * K


def _seg_reduce(x, op):
    return jnp.concatenate(
        [op(x[:, K * c:K * (c + 1)], axis=1, keepdims=True) for c in range(C)],
        axis=1)


def _seg_broadcast(m, rows):
    return jnp.concatenate(
        [jnp.broadcast_to(m[:, c:c + 1], (rows, K)) for c in range(C)], axis=1)


def _num_body(em_ref, ftag_ref, ts_ref, trans_ref, start_ref, end_ref, out_ref):
    L = ts_ref[0, 0, 0]
    trow = jax.lax.broadcasted_iota(jnp.int32, (T, 1), 0)
    maskT = (trow < L).astype(jnp.float32)
    startm = (trow == 0).astype(jnp.float32)
    endm = (trow == L - 1).astype(jnp.float32)
    prevv = jnp.logical_and(trow >= 1, trow < L).astype(jnp.float32)

    lane = jax.lax.broadcasted_iota(jnp.int32, (T, CK), 1)
    tsel = _seg_broadcast(ftag_ref[0], T)
    oh = (lane == tsel).astype(jnp.float32)
    ohp = jnp.concatenate([jnp.zeros((1, CK), jnp.float32), oh[:-1]], axis=0)
    ohp = ohp * prevv

    gval = em_ref[0] * maskT + start_ref[...] * startm + end_ref[...] * endm
    R = jnp.dot(ohp, trans_ref[...], preferred_element_type=jnp.float32)
    contrib = oh * (gval + R)
    tot = jnp.sum(contrib, axis=0, keepdims=True)
    out_ref[...] = _seg_reduce(tot, jnp.sum).reshape(1, 1, C)


def _den_body(em_ref, trans_ref, start_ref, end_ref, ts_ref, out_ref):
    ri = jax.lax.broadcasted_iota(jnp.int32, (CK, CK), 0) // K
    ci = jax.lax.broadcasted_iota(jnp.int32, (CK, CK), 1) // K
    E = jnp.where(ri == ci, jnp.exp(trans_ref[...]), 0.0)

    ts = ts_ref[...]
    alpha0 = start_ref[...] + em_ref[0]

    def step(t, alpha):
        m = _seg_reduce(alpha, jnp.max)
        mb = _seg_broadcast(m, B)
        p = jnp.exp(alpha - mb)
        v = jnp.dot(p, E, preferred_element_type=jnp.float32)
        a_new = jnp.log(v) + mb + em_ref[t]
        return jnp.where(t < ts, a_new, alpha)

    alpha = jax.lax.fori_loop(1, T, step, alpha0)
    af = alpha + end_ref[...]
    m = _seg_reduce(af, jnp.max)
    s = _seg_reduce(jnp.exp(af - _seg_broadcast(m, B)), jnp.sum)
    out_ref[...] = jnp.log(s) + m


@jax.jit
def kernel(emissions, tags, token_sizes, transitions, start_transitions,
           end_transitions):
    f32 = jnp.float32
    em3 = emissions.reshape(T, B, CK).astype(f32)
    emT = jnp.transpose(emissions, (1, 0, 2, 3)).reshape(B, T, CK).astype(f32)
    ftag = (tags.astype(jnp.int32) +
            (K * jnp.arange(C, dtype=jnp.int32))[None, None, :])
    ftagT = jnp.transpose(ftag, (1, 0, 2))
    tsB = token_sizes.astype(jnp.int32).reshape(B, 1)
    tsB3 = tsB.reshape(B, 1, 1)
    transblk = jax.scipy.linalg.block_diag(
        *[transitions[c].astype(f32) for c in range(C)])
    startblk = start_transitions.reshape(1, CK).astype(f32)
    endblk = end_transitions.reshape(1, CK).astype(f32)

    num = pl.pallas_call(
        _num_body,
        grid=(B,),
        in_specs=[
            pl.BlockSpec((1, T, CK), lambda b: (b, 0, 0)),
            pl.BlockSpec((1, T, C), lambda b: (b, 0, 0)),
            pl.BlockSpec((1, 1, 1), lambda b: (b, 0, 0)),
            pl.BlockSpec((CK, CK), lambda b: (0, 0)),
            pl.BlockSpec((1, CK), lambda b: (0, 0)),
            pl.BlockSpec((1, CK), lambda b: (0, 0)),
        ],
        out_specs=pl.BlockSpec((1, 1, C), lambda b: (b, 0, 0)),
        out_shape=jax.ShapeDtypeStruct((B, 1, C), f32),
    )(emT.reshape(B, T, CK), ftagT.reshape(B, T, C), tsB3, transblk, startblk,
      endblk)

    den = pl.pallas_call(
        _den_body,
        in_specs=[
            pl.BlockSpec((T, B, CK), lambda: (0, 0, 0)),
            pl.BlockSpec((CK, CK), lambda: (0, 0)),
            pl.BlockSpec((1, CK), lambda: (0, 0)),
            pl.BlockSpec((1, CK), lambda: (0, 0)),
            pl.BlockSpec((B, 1), lambda: (0, 0)),
        ],
        out_specs=pl.BlockSpec((B, C), lambda: (0, 0)),
        out_shape=jax.ShapeDtypeStruct((B, C), f32),
    )(em3, transblk, startblk, endblk, tsB)

    return num.reshape(B, C) - den


# speedup vs baseline: 89.5840x; 2.0065x over previous
import jax
import jax.numpy as jnp
from jax.experimental import pallas as pl
from jax.experimental.pallas import tpu as pltpu

T, B, C, K = 512, 16, 4, 64
CK = C * K
H = 2 * K


def _body(nwin_ref, em_ref, ftag_ref, ts_ref, trans_ref, start_ref, end_ref,
          out_ref, xem_lo_ref, xem_hi_ref):
    ts = ts_ref[...]

    def blk2(c0):
        z = jnp.zeros((K, K), jnp.float32)
        top = jnp.concatenate([jnp.exp(trans_ref[c0]), z], axis=1)
        bot = jnp.concatenate([z, jnp.exp(trans_ref[c0 + 1])], axis=1)
        return jnp.concatenate([top, bot], axis=0).astype(jnp.bfloat16)

    E_lo, E_hi = blk2(0), blk2(2)
    xem_lo_ref[...] = jnp.exp(em_ref[:, :, :H])
    xem_hi_ref[...] = jnp.exp(em_ref[:, :, H:])

    def init(sl):
        a0 = start_ref[:, sl] + em_ref[0, :, sl]
        m0 = jnp.max(a0, axis=1, keepdims=True)
        return jnp.exp(a0 - m0), m0

    u_lo, M_lo = init(slice(0, H))
    u_hi, M_hi = init(slice(H, CK))

    def window(i, carry):
        u_lo, u_hi, M_lo, M_hi = carry
        t0 = 1 + 4 * i
        for j in range(4):
            t = t0 + j
            tidx = jnp.minimum(t, T - 1)
            mask = t < ts
            v_lo = jnp.dot(u_lo.astype(jnp.bfloat16), E_lo,
                           preferred_element_type=jnp.float32)
            v_hi = jnp.dot(u_hi.astype(jnp.bfloat16), E_hi,
                           preferred_element_type=jnp.float32)
            u_lo = jnp.where(mask, v_lo * xem_lo_ref[tidx], u_lo)
            u_hi = jnp.where(mask, v_hi * xem_hi_ref[tidx], u_hi)
        m_lo = jnp.max(u_lo, axis=1, keepdims=True)
        m_hi = jnp.max(u_hi, axis=1, keepdims=True)
        u_lo = u_lo * (1.0 / m_lo)
        u_hi = u_hi * (1.0 / m_hi)
        return (u_lo, u_hi, M_lo + jnp.log(m_lo), M_hi + jnp.log(m_hi))

    carry = jax.lax.fori_loop(0, nwin_ref[0, 0], window,
                              (u_lo, u_hi, M_lo, M_hi))
    u_lo, u_hi, M_lo, M_hi = carry
    q_lo = u_lo * jnp.exp(end_ref[:, :H])
    q_hi = u_hi * jnp.exp(end_ref[:, H:])
    den = jnp.concatenate(
        [jnp.log(jnp.sum(q_lo[:, :K], axis=1, keepdims=True)) + M_lo,
         jnp.log(jnp.sum(q_lo[:, K:], axis=1, keepdims=True)) + M_lo,
         jnp.log(jnp.sum(q_hi[:, :K], axis=1, keepdims=True)) + M_hi,
         jnp.log(jnp.sum(q_hi[:, K:], axis=1, keepdims=True)) + M_hi],
        axis=1)

    trow = jax.lax.broadcasted_iota(jnp.int32, (T, B, 1), 0)
    tsb = ts.reshape(1, B, 1)
    maskT = (trow < tsb).astype(jnp.float32)
    startm = (trow == 0).astype(jnp.float32)
    endm = (trow == tsb - 1).astype(jnp.float32)

    lane = jax.lax.broadcasted_iota(jnp.int32, (T, B, CK), 2)
    tsel = jnp.concatenate(
        [jnp.broadcast_to(ftag_ref[:, :, c:c + 1], (T, B, K))
         for c in range(C)], axis=2)
    oh = (lane == tsel).astype(jnp.bfloat16)
    ohp = jnp.concatenate(
        [jnp.zeros((1, B, CK), jnp.bfloat16), oh[:-1]], axis=0)
    transblk = jnp.concatenate(
        [jnp.concatenate(
            [trans_ref[c] if c == r else jnp.zeros((K, K), jnp.float32)
             for c in range(C)], axis=1) for r in range(C)],
        axis=0).astype(jnp.bfloat16)
    R = jnp.dot(ohp.reshape(T * B, CK), transblk,
                preferred_element_type=jnp.float32).reshape(T, B, CK)
    prevv = jnp.logical_and(trow >= 1, trow < tsb).astype(jnp.float32)
    gval = (em_ref[...] * maskT + start_ref[...].reshape(1, 1, CK) * startm +
            end_ref[...].reshape(1, 1, CK) * endm + R * prevv)
    contrib = oh.astype(jnp.float32) * gval
    tot = jnp.sum(contrib, axis=0)
    num = jnp.concatenate(
        [jnp.sum(tot[:, K * c:K * (c + 1)], axis=1, keepdims=True)
         for c in range(C)], axis=1)

    out_ref[...] = num - den


@jax.jit
def kernel(emissions, tags, token_sizes, transitions, start_transitions,
           end_transitions):
    f32 = jnp.float32
    em3 = emissions.reshape(T, B, CK).astype(f32)
    ftag = (tags.astype(jnp.int32) +
            (K * jnp.arange(C, dtype=jnp.int32))[None, None, :])
    tsB = token_sizes.astype(jnp.int32).reshape(B, 1)
    startblk = start_transitions.reshape(1, CK).astype(f32)
    endblk = end_transitions.reshape(1, CK).astype(f32)
    nwin = ((jnp.max(token_sizes.astype(jnp.int32)) + 2) // 4).reshape(1, 1)

    return pl.pallas_call(
        _body,
        in_specs=[
            pl.BlockSpec(memory_space=pltpu.SMEM),
            pl.BlockSpec((T, B, CK), lambda: (0, 0, 0)),
            pl.BlockSpec((T, B, C), lambda: (0, 0, 0)),
            pl.BlockSpec((B, 1), lambda: (0, 0)),
            pl.BlockSpec((C, K, K), lambda: (0, 0, 0)),
            pl.BlockSpec((1, CK), lambda: (0, 0)),
            pl.BlockSpec((1, CK), lambda: (0, 0)),
        ],
        out_specs=pl.BlockSpec((B, C), lambda: (0, 0)),
        out_shape=jax.ShapeDtypeStruct((B, C), f32),
        scratch_shapes=[pltpu.VMEM((T, B, H), f32) for _ in range(2)],
    )(nwin, em3, ftag, tsB, transitions.astype(f32), startblk, endblk)


# speedup vs baseline: 124.9400x; 1.3947x over previous
import jax
import jax.numpy as jnp
from jax.experimental import pallas as pl
from jax.experimental.pallas import tpu as pltpu

T, B, C, K = 512, 16, 4, 64
CK = C * K
H = 2 * K


def _body(scal_ref, em_ref, ftag_ref, ts_ref, trans_ref, transT_ref,
          start_ref, end_ref, out_ref, xem_lo_ref, xem_hi_ref):
    ts = ts_ref[...]
    nwmax = scal_ref[0, 0]
    mid = scal_ref[0, 1]
    maxL = scal_ref[0, 2]

    def blk2(ref, c0):
        z = jnp.zeros((K, K), jnp.float32)
        top = jnp.concatenate([jnp.exp(ref[c0]), z], axis=1)
        bot = jnp.concatenate([z, jnp.exp(ref[c0 + 1])], axis=1)
        return jnp.concatenate([top, bot], axis=0).astype(jnp.bfloat16)

    E_lo, E_hi = blk2(trans_ref, 0), blk2(trans_ref, 2)
    ET_lo, ET_hi = blk2(transT_ref, 0), blk2(transT_ref, 2)
    xem_lo_ref[...] = jnp.exp(em_ref[:, :, :H])
    xem_hi_ref[...] = jnp.exp(em_ref[:, :, H:])

    def init(sl):
        a0 = start_ref[:, sl] + em_ref[0, :, sl]
        m0 = jnp.max(a0, axis=1, keepdims=True)
        return jnp.exp(a0 - m0), m0

    ua_lo, Ma_lo = init(slice(0, H))
    ua_hi, Ma_hi = init(slice(H, CK))
    ub_lo = jnp.broadcast_to(jnp.exp(end_ref[:, :H]), (B, H))
    ub_hi = jnp.broadcast_to(jnp.exp(end_ref[:, H:]), (B, H))
    Mb_lo = jnp.zeros((B, 1), jnp.float32)
    Mb_hi = jnp.zeros((B, 1), jnp.float32)

    def window(w, carry):
        ua_lo, ua_hi, ub_lo, ub_hi, Ma_lo, Ma_hi, Mb_lo, Mb_hi = carry
        for j in range(4):
            tf = 1 + 4 * w + j
            tfi = jnp.minimum(tf, T - 1)
            mf = jnp.logical_and(tf < ts, tf <= mid)
            va_lo = jnp.dot(ua_lo.astype(jnp.bfloat16), E_lo,
                            preferred_element_type=jnp.float32)
            va_hi = jnp.dot(ua_hi.astype(jnp.bfloat16), E_hi,
                            preferred_element_type=jnp.float32)
            tb = maxL - 2 - 4 * w - j
            tbi = jnp.clip(tb + 1, 0, T - 1)
            mb = jnp.logical_and(tb >= mid, tb < ts - 1)
            vb_lo = jnp.dot((ub_lo * xem_lo_ref[tbi]).astype(jnp.bfloat16),
                            ET_lo, preferred_element_type=jnp.float32)
            vb_hi = jnp.dot((ub_hi * xem_hi_ref[tbi]).astype(jnp.bfloat16),
                            ET_hi, preferred_element_type=jnp.float32)
            ua_lo = jnp.where(mf, va_lo * xem_lo_ref[tfi], ua_lo)
            ua_hi = jnp.where(mf, va_hi * xem_hi_ref[tfi], ua_hi)
            ub_lo = jnp.where(mb, vb_lo, ub_lo)
            ub_hi = jnp.where(mb, vb_hi, ub_hi)
        outs = []
        for u, M in ((ua_lo, Ma_lo), (ua_hi, Ma_hi), (ub_lo, Mb_lo),
                     (ub_hi, Mb_hi)):
            m = jnp.max(u, axis=1, keepdims=True)
            outs.append((u * (1.0 / m), M + jnp.log(m)))
        return (outs[0][0], outs[1][0], outs[2][0], outs[3][0],
                outs[0][1], outs[1][1], outs[2][1], outs[3][1])

    carry = jax.lax.fori_loop(0, nwmax, window,
                              (ua_lo, ua_hi, ub_lo, ub_hi,
                               Ma_lo, Ma_hi, Mb_lo, Mb_hi))
    ua_lo, ua_hi, ub_lo, ub_hi, Ma_lo, Ma_hi, Mb_lo, Mb_hi = carry
    q_lo = ua_lo * ub_lo
    q_hi = ua_hi * ub_hi
    den = jnp.concatenate(
        [jnp.log(jnp.sum(q_lo[:, :K], axis=1, keepdims=True)) + Ma_lo + Mb_lo,
         jnp.log(jnp.sum(q_lo[:, K:], axis=1, keepdims=True)) + Ma_lo + Mb_lo,
         jnp.log(jnp.sum(q_hi[:, :K], axis=1, keepdims=True)) + Ma_hi + Mb_hi,
         jnp.log(jnp.sum(q_hi[:, K:], axis=1, keepdims=True)) + Ma_hi + Mb_hi],
        axis=1)

    trow = jax.lax.broadcasted_iota(jnp.int32, (T, B, 1), 0)
    tsb = ts.reshape(1, B, 1)
    maskT = (trow < tsb).astype(jnp.float32)
    startm = (trow == 0).astype(jnp.float32)
    endm = (trow == tsb - 1).astype(jnp.float32)

    lane = jax.lax.broadcasted_iota(jnp.int32, (T, B, CK), 2)
    tsel = jnp.concatenate(
        [jnp.broadcast_to(ftag_ref[:, :, c:c + 1], (T, B, K))
         for c in range(C)], axis=2)
    oh = (lane == tsel).astype(jnp.bfloat16)
    ohp = jnp.concatenate(
        [jnp.zeros((1, B, CK), jnp.bfloat16), oh[:-1]], axis=0)
    transblk = jnp.concatenate(
        [jnp.concatenate(
            [trans_ref[c] if c == r else jnp.zeros((K, K), jnp.float32)
             for c in range(C)], axis=1) for r in range(C)],
        axis=0).astype(jnp.bfloat16)
    R = jnp.dot(ohp.reshape(T * B, CK), transblk,
                preferred_element_type=jnp.float32).reshape(T, B, CK)
    prevv = jnp.logical_and(trow >= 1, trow < tsb).astype(jnp.float32)
    gval = (em_ref[...] * maskT + start_ref[...].reshape(1, 1, CK) * startm +
            end_ref[...].reshape(1, 1, CK) * endm + R * prevv)
    contrib = oh.astype(jnp.float32) * gval
    tot = jnp.sum(contrib, axis=0)
    num = jnp.concatenate(
        [jnp.sum(tot[:, K * c:K * (c + 1)], axis=1, keepdims=True)
         for c in range(C)], axis=1)

    out_ref[...] = num - den


@jax.jit
def kernel(emissions, tags, token_sizes, transitions, start_transitions,
           end_transitions):
    f32 = jnp.float32
    em3 = emissions.reshape(T, B, CK).astype(f32)
    ftag = (tags.astype(jnp.int32) +
            (K * jnp.arange(C, dtype=jnp.int32))[None, None, :])
    tsB = token_sizes.astype(jnp.int32).reshape(B, 1)
    startblk = start_transitions.reshape(1, CK).astype(f32)
    endblk = end_transitions.reshape(1, CK).astype(f32)
    maxL = jnp.max(token_sizes.astype(jnp.int32))
    nfwin = ((maxL - 1) // 2 + 3) // 4
    mid = 4 * nfwin
    nbwin = (jnp.maximum(maxL - 1 - mid, 0) + 3) // 4
    nwmax = jnp.maximum(nfwin, nbwin)
    scal = jnp.stack([nwmax, mid, maxL]).reshape(1, 3)

    return pl.pallas_call(
        _body,
        in_specs=[
            pl.BlockSpec(memory_space=pltpu.SMEM),
            pl.BlockSpec((T, B, CK), lambda: (0, 0, 0)),
            pl.BlockSpec((T, B, C), lambda: (0, 0, 0)),
            pl.BlockSpec((B, 1), lambda: (0, 0)),
            pl.BlockSpec((C, K, K), lambda: (0, 0, 0)),
            pl.BlockSpec((C, K, K), lambda: (0, 0, 0)),
            pl.BlockSpec((1, CK), lambda: (0, 0)),
            pl.BlockSpec((1, CK), lambda: (0, 0)),
        ],
        out_specs=pl.BlockSpec((B, C), lambda: (0, 0)),
        out_shape=jax.ShapeDtypeStruct((B, C), f32),
        scratch_shapes=[pltpu.VMEM((T, B, H), f32) for _ in range(2)],
    )(scal, em3, ftag, tsB, transitions.astype(f32),
      jnp.transpose(transitions, (0, 2, 1)).astype(f32), startblk, endblk)
